# baseline (device time: 52764 ns/iter reference)
import jax
import jax.numpy as jnp
from jax import lax
from jax.experimental import pallas as pl
from jax.experimental.pallas import tpu as pltpu

N_DEV = 16


def kernel(x, w_mat):
    m_per, k = x.shape
    _, n = w_mat.shape
    n_per = n // N_DEV

    def body(x_ref, w_ref, out_ref, y_ref, x_bf_ref, amax_ref, gather_ref,
             data_send_sems, data_recv_sems, amax_send_sems, amax_recv_sems):
        my = lax.axis_index("i")

        barrier_sem = pltpu.get_barrier_semaphore()
        for j in range(1, N_DEV):
            t = lax.rem(my + j, N_DEV)
            pl.semaphore_signal(
                barrier_sem, inc=1,
                device_id=(t,), device_id_type=pl.DeviceIdType.MESH,
            )
        pl.semaphore_wait(barrier_sem, N_DEV - 1)

        x_bf_ref[...] = x_ref[...].astype(jnp.bfloat16)
        for t in range(N_DEV):
            y_ref[t] = jnp.dot(
                x_bf_ref[...],
                w_ref[:, t * n_per:(t + 1) * n_per].astype(jnp.bfloat16),
                preferred_element_type=jnp.float32,
            )
        local_amax = jnp.max(jnp.abs(y_ref[...]))
        amax_ref[...] = jnp.broadcast_to(local_amax, (1, 128))
        gather_ref[pl.ds(my, 1), :] = amax_ref[...]

        out_ref[pl.ds(my * m_per, m_per), :] = y_ref[my]

        data_rdmas = []
        amax_rdmas = []
        for j in range(1, N_DEV):
            t = lax.rem(my + j, N_DEV)
            rdma = pltpu.make_async_remote_copy(
                src_ref=y_ref.at[t],
                dst_ref=out_ref.at[pl.ds(my * m_per, m_per), :],
                send_sem=data_send_sems.at[j],
                recv_sem=data_recv_sems.at[j],
                device_id=(t,),
                device_id_type=pl.DeviceIdType.MESH,
            )
            rdma.start()
            data_rdmas.append(rdma)
            a_rdma = pltpu.make_async_remote_copy(
                src_ref=amax_ref,
                dst_ref=gather_ref.at[pl.ds(my, 1), :],
                send_sem=amax_send_sems.at[j],
                recv_sem=amax_recv_sems.at[j],
                device_id=(t,),
                device_id_type=pl.DeviceIdType.MESH,
            )
            a_rdma.start()
            amax_rdmas.append(a_rdma)

        for rdma in data_rdmas:
            rdma.wait()
        for a_rdma in amax_rdmas:
            a_rdma.wait()

        scale = jnp.max(gather_ref[...]) / 127.0
        q = jnp.clip(jnp.round(out_ref[...] / scale), -127.0, 127.0)
        out_ref[...] = q * scale

    return pl.pallas_call(
        body,
        out_shape=jax.ShapeDtypeStruct((N_DEV * m_per, n_per), jnp.float32),
        in_specs=[
            pl.BlockSpec(memory_space=pltpu.VMEM),
            pl.BlockSpec(memory_space=pltpu.VMEM),
        ],
        out_specs=pl.BlockSpec(memory_space=pltpu.VMEM),
        scratch_shapes=[
            pltpu.VMEM((N_DEV, m_per, n_per), jnp.float32),
            pltpu.VMEM((m_per, k), jnp.bfloat16),
            pltpu.VMEM((1, 128), jnp.float32),
            pltpu.VMEM((N_DEV, 128), jnp.float32),
            pltpu.SemaphoreType.DMA((N_DEV,)),
            pltpu.SemaphoreType.DMA((N_DEV,)),
            pltpu.SemaphoreType.DMA((N_DEV,)),
            pltpu.SemaphoreType.DMA((N_DEV,)),
        ],
        compiler_params=pltpu.CompilerParams(
            collective_id=0, vmem_limit_bytes=100 * 1024 * 1024,
        ),
    )(x, w_mat)


# device time: 36017 ns/iter; 1.4650x vs baseline; 1.4650x over previous
import os

import jax
import jax.numpy as jnp
from jax import lax
from jax.experimental import pallas as pl
from jax.experimental.pallas import tpu as pltpu

N_DEV = 16

_VARIANT = os.environ.get("KVARIANT", "full")


def kernel(x, w_mat):
    m_per, k = x.shape
    _, n = w_mat.shape
    n_per = n // N_DEV

    def body(x_ref, w_hbm, out_ref, x_bf_ref, w_buf, y_buf, recv_buf,
             amax_ref, gather_ref, w_sems,
             data_send_sems, data_recv_sems, amax_send_sems, amax_recv_sems):
        my = lax.axis_index("i")

        def w_dma(j):
            t = lax.rem(my + j, N_DEV)
            return pltpu.make_async_copy(
                w_hbm.at[:, pl.ds(t * n_per, n_per)],
                w_buf.at[j % 2],
                w_sems.at[j % 2],
            )

        w_dma(1).start()
        w_dma(2).start()
        x_bf_ref[...] = x_ref[...].astype(jnp.bfloat16)

        barrier_sem = pltpu.get_barrier_semaphore()
        for j in range(1, N_DEV):
            t = lax.rem(my + j, N_DEV)
            pl.semaphore_signal(
                barrier_sem, inc=1,
                device_id=(t,), device_id_type=pl.DeviceIdType.MESH,
            )
        pl.semaphore_wait(barrier_sem, N_DEV - 1)

        data_rdmas = []
        amax_run = None
        for j in range(1, N_DEV + 1):
            t = lax.rem(my + j, N_DEV)
            w_dma(j).wait()
            if _VARIANT == "nocompute":
                y = jnp.broadcast_to(x_ref[0, 0], (m_per, n_per))
            else:
                y = jnp.dot(
                    x_bf_ref[...], w_buf[j % 2].astype(jnp.bfloat16),
                    preferred_element_type=jnp.float32,
                )
            if j + 2 <= N_DEV:
                w_dma(j + 2).start()
            a = jnp.max(jnp.abs(y))
            amax_run = a if amax_run is None else jnp.maximum(amax_run, a)
            if j < N_DEV:
                y_buf[pl.ds(t, 1)] = y.astype(jnp.bfloat16)[None]
                rdma = pltpu.make_async_remote_copy(
                    src_ref=y_buf.at[t],
                    dst_ref=recv_buf.at[pl.ds(my * m_per, m_per), :],
                    send_sem=data_send_sems.at[j],
                    recv_sem=data_recv_sems.at[j],
                    device_id=(t,),
                    device_id_type=pl.DeviceIdType.MESH,
                )
                if _VARIANT != "nocomm":
                    rdma.start()
                    data_rdmas.append(rdma)
            else:
                recv_buf[pl.ds(my * m_per, m_per), :] = y.astype(jnp.bfloat16)

        amax_ref[...] = jnp.broadcast_to(amax_run, (1, 128))
        gather_ref[pl.ds(my, 1), :] = amax_ref[...]
        amax_rdmas = []
        if _VARIANT != "nocomm":
            for j in range(1, N_DEV):
                t = lax.rem(my + j, N_DEV)
                a_rdma = pltpu.make_async_remote_copy(
                    src_ref=amax_ref,
                    dst_ref=gather_ref.at[pl.ds(my, 1), :],
                    send_sem=amax_send_sems.at[j],
                    recv_sem=amax_recv_sems.at[j],
                    device_id=(t,),
                    device_id_type=pl.DeviceIdType.MESH,
                )
                a_rdma.start()
                amax_rdmas.append(a_rdma)

        for rdma in data_rdmas:
            rdma.wait()
        for a_rdma in amax_rdmas:
            a_rdma.wait()

        scale = jnp.max(gather_ref[...]) / 127.0
        yf = recv_buf[...].astype(jnp.float32)
        q = jnp.clip(jnp.round(yf / scale), -127.0, 127.0)
        out_ref[...] = q * scale

    return pl.pallas_call(
        body,
        out_shape=jax.ShapeDtypeStruct((N_DEV * m_per, n_per), jnp.float32),
        in_specs=[
            pl.BlockSpec(memory_space=pltpu.VMEM),
            pl.BlockSpec(memory_space=pl.ANY),
        ],
        out_specs=pl.BlockSpec(memory_space=pltpu.VMEM),
        scratch_shapes=[
            pltpu.VMEM((m_per, k), jnp.bfloat16),
            pltpu.VMEM((2, k, n_per), jnp.float32),
            pltpu.VMEM((N_DEV, m_per, n_per), jnp.bfloat16),
            pltpu.VMEM((N_DEV * m_per, n_per), jnp.bfloat16),
            pltpu.VMEM((1, 128), jnp.float32),
            pltpu.VMEM((N_DEV, 128), jnp.float32),
            pltpu.SemaphoreType.DMA((2,)),
            pltpu.SemaphoreType.DMA((N_DEV,)),
            pltpu.SemaphoreType.DMA((N_DEV,)),
            pltpu.SemaphoreType.DMA((N_DEV,)),
            pltpu.SemaphoreType.DMA((N_DEV,)),
        ],
        compiler_params=pltpu.CompilerParams(
            collective_id=0, vmem_limit_bytes=100 * 1024 * 1024,
        ),
    )(x, w_mat)


# device time: 32240 ns/iter; 1.6366x vs baseline; 1.1172x over previous
import os

import jax
import jax.numpy as jnp
from jax import lax
from jax.experimental import pallas as pl
from jax.experimental.pallas import tpu as pltpu

N_DEV = 16

_VARIANT = os.environ.get("KVARIANT", "full")


def kernel(x, w_mat):
    m_per, k = x.shape
    _, n = w_mat.shape
    n_per = n // N_DEV

    N_SLOT = 4

    def body(x_hbm, w_hbm, out_ref, x_ref, x_bf_ref, w_buf, y_buf, recv_buf,
             amax_ref, gather_ref, x_sem, w_sems,
             data_send_sems, data_recv_sems, amax_send_sems, amax_recv_sems):
        my = lax.axis_index("i")

        def w_dma(j):
            t = lax.rem(my + j, N_DEV)
            return pltpu.make_async_copy(
                w_hbm.at[:, pl.ds(t * n_per, n_per)],
                w_buf.at[j % N_SLOT],
                w_sems.at[j % N_SLOT],
            )

        x_dma = pltpu.make_async_copy(x_hbm, x_ref, x_sem)
        x_dma.start()
        for j in range(1, 1 + N_SLOT):
            w_dma(j).start()
        x_dma.wait()
        x_bf_ref[...] = x_ref[...].astype(jnp.bfloat16)

        barrier_sem = pltpu.get_barrier_semaphore()
        for j in range(1, N_DEV):
            t = lax.rem(my + j, N_DEV)
            pl.semaphore_signal(
                barrier_sem, inc=1,
                device_id=(t,), device_id_type=pl.DeviceIdType.MESH,
            )
        pl.semaphore_wait(barrier_sem, N_DEV - 1)

        data_rdmas = []
        amax_run = None
        for j in range(1, N_DEV + 1):
            t = lax.rem(my + j, N_DEV)
            w_dma(j).wait()
            if _VARIANT == "nocompute":
                y = jnp.broadcast_to(x_ref[0, 0], (m_per, n_per))
            else:
                y = jnp.dot(
                    x_bf_ref[...], w_buf[j % N_SLOT].astype(jnp.bfloat16),
                    preferred_element_type=jnp.float32,
                )
            if j + N_SLOT <= N_DEV:
                w_dma(j + N_SLOT).start()
            a = jnp.max(jnp.abs(y))
            amax_run = a if amax_run is None else jnp.maximum(amax_run, a)
            if j < N_DEV:
                y_buf[pl.ds(t, 1)] = y.astype(jnp.bfloat16)[None]
                rdma = pltpu.make_async_remote_copy(
                    src_ref=y_buf.at[t],
                    dst_ref=recv_buf.at[pl.ds(my * m_per, m_per), :],
                    send_sem=data_send_sems.at[j],
                    recv_sem=data_recv_sems.at[j],
                    device_id=(t,),
                    device_id_type=pl.DeviceIdType.MESH,
                )
                if _VARIANT != "nocomm":
                    rdma.start()
                    data_rdmas.append(rdma)
            else:
                recv_buf[pl.ds(my * m_per, m_per), :] = y.astype(jnp.bfloat16)

        amax_ref[...] = jnp.broadcast_to(amax_run, (1, 128))
        gather_ref[pl.ds(my, 1), :] = amax_ref[...]
        amax_rdmas = []
        if _VARIANT != "nocomm":
            for j in range(1, N_DEV):
                t = lax.rem(my + j, N_DEV)
                a_rdma = pltpu.make_async_remote_copy(
                    src_ref=amax_ref,
                    dst_ref=gather_ref.at[pl.ds(my, 1), :],
                    send_sem=amax_send_sems.at[j],
                    recv_sem=amax_recv_sems.at[j],
                    device_id=(t,),
                    device_id_type=pl.DeviceIdType.MESH,
                )
                a_rdma.start()
                amax_rdmas.append(a_rdma)

        for rdma in data_rdmas:
            rdma.wait()
        for a_rdma in amax_rdmas:
            a_rdma.wait()

        scale = jnp.max(gather_ref[...]) / 127.0
        yf = recv_buf[...].astype(jnp.float32)
        q = jnp.clip(jnp.round(yf / scale), -127.0, 127.0)
        out_ref[...] = q * scale

    return pl.pallas_call(
        body,
        out_shape=jax.ShapeDtypeStruct((N_DEV * m_per, n_per), jnp.float32),
        in_specs=[
            pl.BlockSpec(memory_space=pl.ANY),
            pl.BlockSpec(memory_space=pl.ANY),
        ],
        out_specs=pl.BlockSpec(memory_space=pltpu.VMEM),
        scratch_shapes=[
            pltpu.VMEM((m_per, k), jnp.float32),
            pltpu.VMEM((m_per, k), jnp.bfloat16),
            pltpu.VMEM((N_SLOT, k, n_per), jnp.float32),
            pltpu.VMEM((N_DEV, m_per, n_per), jnp.bfloat16),
            pltpu.VMEM((N_DEV * m_per, n_per), jnp.bfloat16),
            pltpu.VMEM((1, 128), jnp.float32),
            pltpu.VMEM((N_DEV, 128), jnp.float32),
            pltpu.SemaphoreType.DMA,
            pltpu.SemaphoreType.DMA((N_SLOT,)),
            pltpu.SemaphoreType.DMA((N_DEV,)),
            pltpu.SemaphoreType.DMA((N_DEV,)),
            pltpu.SemaphoreType.DMA((N_DEV,)),
            pltpu.SemaphoreType.DMA((N_DEV,)),
        ],
        compiler_params=pltpu.CompilerParams(
            collective_id=0, vmem_limit_bytes=100 * 1024 * 1024,
        ),
    )(x, w_mat)


# device time: 30714 ns/iter; 1.7179x vs baseline; 1.0497x over previous
import os

import jax
import jax.numpy as jnp
from jax import lax
from jax.experimental import pallas as pl
from jax.experimental.pallas import tpu as pltpu

N_DEV = 16

_VARIANT = os.environ.get("KVARIANT", "full")


def kernel(x, w_mat):
    m_per, k = x.shape
    _, n = w_mat.shape
    n_per = n // N_DEV

    N_TILE = 4
    n_tile = n // N_TILE

    def body(x_hbm, w_hbm, out_ref, x_ref, x_bf_ref, w_buf, y_buf, recv_buf,
             amax_ref, gather_ref, x_sem, w_sems,
             data_send_sems, data_recv_sems, amax_send_sems, amax_recv_sems):
        my = lax.axis_index("i")
        my_g = lax.div(my, 4)

        def w_dma(j):
            tt = lax.rem(my_g + j, N_TILE)
            return pltpu.make_async_copy(
                w_hbm.at[:, pl.ds(tt * n_tile, n_tile)],
                w_buf.at[(j - 1) % 2],
                w_sems.at[(j - 1) % 2],
            )

        x_dma = pltpu.make_async_copy(x_hbm, x_ref, x_sem)
        x_dma.start()
        w_dma(1).start()
        w_dma(2).start()
        x_dma.wait()
        x_bf_ref[...] = x_ref[...].astype(jnp.bfloat16)

        if _VARIANT != "streamonly":
            barrier_sem = pltpu.get_barrier_semaphore()
            for j in range(1, N_DEV):
                t = lax.rem(my + j, N_DEV)
                pl.semaphore_signal(
                    barrier_sem, inc=1,
                    device_id=(t,), device_id_type=pl.DeviceIdType.MESH,
                )
            pl.semaphore_wait(barrier_sem, N_DEV - 1)

        data_rdmas = []
        amax_run = None
        for j in range(1, N_TILE + 1):
            tt = lax.rem(my_g + j, N_TILE)
            w_dma(j).wait()
            if _VARIANT == "nocompute":
                y = jnp.broadcast_to(x_ref[0, 0], (m_per, n_tile))
            else:
                y = jnp.dot(
                    x_bf_ref[...], w_buf[(j - 1) % 2].astype(jnp.bfloat16),
                    preferred_element_type=jnp.float32,
                )
            if j + 2 <= N_TILE:
                w_dma(j + 2).start()
            a = jnp.max(jnp.abs(y))
            amax_run = a if amax_run is None else jnp.maximum(amax_run, a)
            yb = y.astype(jnp.bfloat16)
            for u in range(N_TILE):
                y_buf[4 * (j - 1) + u] = yb[:, u * n_per:(u + 1) * n_per]
            for c in range(N_TILE):
                u = lax.rem(my + c, 4)
                idx = 4 * (j - 1) + u
                if j == N_TILE and c == 0:
                    recv_buf[pl.ds(my * m_per, m_per), :] = y_buf[idx]
                    continue
                t = 4 * tt + u
                k = 4 * (j - 1) + c
                rdma = pltpu.make_async_remote_copy(
                    src_ref=y_buf.at[idx],
                    dst_ref=recv_buf.at[pl.ds(my * m_per, m_per), :],
                    send_sem=data_send_sems.at[k],
                    recv_sem=data_recv_sems.at[k],
                    device_id=(t,),
                    device_id_type=pl.DeviceIdType.MESH,
                )
                if _VARIANT not in ("nocomm", "streamonly"):
                    rdma.start()
                    data_rdmas.append(rdma)

        amax_ref[...] = jnp.broadcast_to(amax_run, (1, 128))
        gather_ref[pl.ds(my, 1), :] = amax_ref[...]
        amax_rdmas = []
        if _VARIANT not in ("nocomm", "streamonly", "noamax"):
            for j in range(1, N_DEV):
                t = lax.rem(my + j, N_DEV)
                a_rdma = pltpu.make_async_remote_copy(
                    src_ref=amax_ref,
                    dst_ref=gather_ref.at[pl.ds(my, 1), :],
                    send_sem=amax_send_sems.at[j],
                    recv_sem=amax_recv_sems.at[j],
                    device_id=(t,),
                    device_id_type=pl.DeviceIdType.MESH,
                )
                a_rdma.start()
                amax_rdmas.append(a_rdma)

        for rdma in data_rdmas:
            rdma.wait()
        for a_rdma in amax_rdmas:
            a_rdma.wait()

        scale = jnp.max(gather_ref[...]) / 127.0
        yf = recv_buf[...].astype(jnp.float32)
        q = jnp.clip(jnp.round(yf / scale), -127.0, 127.0)
        out_ref[...] = q * scale

    return pl.pallas_call(
        body,
        out_shape=jax.ShapeDtypeStruct((N_DEV * m_per, n_per), jnp.float32),
        in_specs=[
            pl.BlockSpec(memory_space=pl.ANY),
            pl.BlockSpec(memory_space=pl.ANY),
        ],
        out_specs=pl.BlockSpec(memory_space=pltpu.VMEM),
        scratch_shapes=[
            pltpu.VMEM((m_per, k), jnp.float32),
            pltpu.VMEM((m_per, k), jnp.bfloat16),
            pltpu.VMEM((2, k, n_tile), jnp.float32),
            pltpu.VMEM((N_DEV, m_per, n_per), jnp.bfloat16),
            pltpu.VMEM((N_DEV * m_per, n_per), jnp.bfloat16),
            pltpu.VMEM((1, 128), jnp.float32),
            pltpu.VMEM((N_DEV, 128), jnp.float32),
            pltpu.SemaphoreType.DMA,
            pltpu.SemaphoreType.DMA((2,)),
            pltpu.SemaphoreType.DMA((N_DEV,)),
            pltpu.SemaphoreType.DMA((N_DEV,)),
            pltpu.SemaphoreType.DMA((N_DEV,)),
            pltpu.SemaphoreType.DMA((N_DEV,)),
        ],
        compiler_params=pltpu.CompilerParams(
            collective_id=0, vmem_limit_bytes=100 * 1024 * 1024,
        ),
    )(x, w_mat)
